# initial kernel scaffold (unmeasured)
import jax
import jax.numpy as jnp
from jax import lax
from jax.experimental import pallas as pl
from jax.experimental.pallas import tpu as pltpu

M = 512
N = 256
G = 2 * M


def kernel(x, dest):
    dest2 = dest.reshape(1, M)

    def body(x_ref, dest_ref, out_ref, xg_ref, dg_ref, send_sems, recv_sems):
        my_x = lax.axis_index("x")
        my_y = lax.axis_index("y")
        other = 1 - my_x

        barrier_sem = pltpu.get_barrier_semaphore()
        pl.semaphore_signal(
            barrier_sem, inc=1,
            device_id=(other, my_y), device_id_type=pl.DeviceIdType.MESH,
        )
        pl.semaphore_wait(barrier_sem, 1)

        xg_ref[pl.ds(my_x * M, M), :] = x_ref[...]
        dg_ref[my_x] = dest_ref[...]

        rdma_x = pltpu.make_async_remote_copy(
            src_ref=x_ref,
            dst_ref=xg_ref.at[pl.ds(my_x * M, M), :],
            send_sem=send_sems.at[0],
            recv_sem=recv_sems.at[0],
            device_id=(other, my_y),
            device_id_type=pl.DeviceIdType.MESH,
        )
        rdma_d = pltpu.make_async_remote_copy(
            src_ref=dest_ref,
            dst_ref=dg_ref.at[my_x],
            send_sem=send_sems.at[1],
            recv_sem=recv_sems.at[1],
            device_id=(other, my_y),
            device_id_type=pl.DeviceIdType.MESH,
        )
        rdma_x.start()
        rdma_d.start()
        rdma_x.wait()
        rdma_d.wait()

        dg = dg_ref[...].reshape(1, G)
        mine = dg == my_x
        mask = mine.astype(jnp.int32)
        c = jnp.cumsum(mask, axis=1) - mask
        k = lax.broadcasted_iota(jnp.int32, (M, G), 0)
        sel = mine & (c == k)
        p = sel.astype(jnp.float32)
        out_ref[...] = jnp.dot(
            p, xg_ref[...], preferred_element_type=jnp.float32
        )

    return pl.pallas_call(
        body,
        out_shape=jax.ShapeDtypeStruct((M, N), jnp.float32),
        in_specs=[
            pl.BlockSpec(memory_space=pltpu.VMEM),
            pl.BlockSpec(memory_space=pltpu.VMEM),
        ],
        out_specs=pl.BlockSpec(memory_space=pltpu.VMEM),
        scratch_shapes=[
            pltpu.VMEM((G, N), jnp.float32),
            pltpu.VMEM((2, 1, M), jnp.int32),
            pltpu.SemaphoreType.DMA((2,)),
            pltpu.SemaphoreType.DMA((2,)),
        ],
        compiler_params=pltpu.CompilerParams(collective_id=0),
    )(x, dest2)


# baseline (device time: 12503 ns/iter reference)
import jax
import jax.numpy as jnp
from jax import lax
from jax.experimental import pallas as pl
from jax.experimental.pallas import tpu as pltpu

M = 512
N = 256
G = 2 * M


def kernel(x, dest):
    dest2 = dest.reshape(1, M)

    def body(x_ref, dest_ref, out_ref, xg_ref, dg_ref, send_sems, recv_sems):
        my_x = lax.axis_index("x")
        my_y = lax.axis_index("y")
        other = 1 - my_x

        barrier_sem = pltpu.get_barrier_semaphore()
        pl.semaphore_signal(
            barrier_sem, inc=1,
            device_id=(other, my_y), device_id_type=pl.DeviceIdType.MESH,
        )
        pl.semaphore_wait(barrier_sem, 1)

        xg_ref[pl.ds(my_x * M, M), :] = x_ref[...]
        dg_ref[my_x] = dest_ref[...]

        rdma_x = pltpu.make_async_remote_copy(
            src_ref=x_ref,
            dst_ref=xg_ref.at[pl.ds(my_x * M, M), :],
            send_sem=send_sems.at[0],
            recv_sem=recv_sems.at[0],
            device_id=(other, my_y),
            device_id_type=pl.DeviceIdType.MESH,
        )
        rdma_d = pltpu.make_async_remote_copy(
            src_ref=dest_ref,
            dst_ref=dg_ref.at[my_x],
            send_sem=send_sems.at[1],
            recv_sem=recv_sems.at[1],
            device_id=(other, my_y),
            device_id_type=pl.DeviceIdType.MESH,
        )
        rdma_x.start()
        rdma_d.start()
        rdma_x.wait()
        rdma_d.wait()

        dg = dg_ref[...].reshape(1, G)
        mine = dg == my_x
        mask = mine.astype(jnp.float32)
        tri = (
            lax.broadcasted_iota(jnp.int32, (G, G), 0)
            < lax.broadcasted_iota(jnp.int32, (G, G), 1)
        ).astype(jnp.float32)
        c = jnp.dot(mask, tri, preferred_element_type=jnp.float32)
        k = lax.broadcasted_iota(jnp.int32, (M, G), 0)
        sel = mine & (c.astype(jnp.int32) == k)
        p = sel.astype(jnp.float32)
        out_ref[...] = jnp.dot(
            p, xg_ref[...], preferred_element_type=jnp.float32
        )

    return pl.pallas_call(
        body,
        out_shape=jax.ShapeDtypeStruct((M, N), jnp.float32),
        in_specs=[
            pl.BlockSpec(memory_space=pltpu.VMEM),
            pl.BlockSpec(memory_space=pltpu.VMEM),
        ],
        out_specs=pl.BlockSpec(memory_space=pltpu.VMEM),
        scratch_shapes=[
            pltpu.VMEM((G, N), jnp.float32),
            pltpu.VMEM((2, 1, M), jnp.int32),
            pltpu.SemaphoreType.DMA((2,)),
            pltpu.SemaphoreType.DMA((2,)),
        ],
        compiler_params=pltpu.CompilerParams(collective_id=0),
    )(x, dest2)


# device time: 10284 ns/iter; 1.2158x vs baseline; 1.2158x over previous
import jax
import jax.numpy as jnp
from jax import lax
from jax.experimental import pallas as pl
from jax.experimental.pallas import tpu as pltpu

M = 512
N = 256
CH = 64
NCH = M // CH


def kernel(x, dest):
    dest2 = dest.reshape(1, M)

    def body(x_ref, dest_ref, out_ref, sbuf_ref, rbuf_ref, send_sems, recv_sems):
        my_x = lax.axis_index("x")
        my_y = lax.axis_index("y")
        other = 1 - my_x

        barrier_sem = pltpu.get_barrier_semaphore()
        pl.semaphore_signal(
            barrier_sem, inc=1,
            device_id=(other, my_y), device_id_type=pl.DeviceIdType.MESH,
        )
        pl.semaphore_wait(barrier_sem, 1)

        dl = dest_ref[...]
        keep = dl == my_x
        maskf = keep.astype(jnp.float32)
        tri = (
            lax.broadcasted_iota(jnp.int32, (M, M), 0)
            < lax.broadcasted_iota(jnp.int32, (M, M), 1)
        ).astype(jnp.float32)
        ck = jnp.dot(maskf, tri, preferred_element_type=jnp.float32)
        ck = ck.astype(jnp.int32)
        il = lax.broadcasted_iota(jnp.int32, (1, M), 1)
        cs = il - ck
        n_keep = jnp.sum(keep.astype(jnp.int32))
        n_mov = M - n_keep
        off = my_x * n_mov

        jrow = lax.broadcasted_iota(jnp.int32, (2 * M, M), 0)
        pk = keep & (jrow == ck + off)
        ps = (~keep) & (jrow == cs + off + M)
        big_p = (pk | ps).astype(jnp.float32)
        both = jnp.dot(big_p, x_ref[...], preferred_element_type=jnp.float32)
        keep_rows = both[:M, :]
        sbuf_ref[...] = both[M:, :]

        nc = (n_mov + CH - 1) // CH

        for k in range(NCH):
            o_send = jnp.where(my_x == 0, CH * k, M - CH * (k + 1))

            @pl.when(k < nc)
            def _():
                rdma = pltpu.make_async_remote_copy(
                    src_ref=sbuf_ref.at[pl.ds(o_send, CH), :],
                    dst_ref=rbuf_ref.at[pl.ds(o_send, CH), :],
                    send_sem=send_sems.at[k],
                    recv_sem=recv_sems.at[k],
                    device_id=(other, my_y),
                    device_id_type=pl.DeviceIdType.MESH,
                )
                rdma.start()

        for k in range(NCH):
            o_send = jnp.where(my_x == 0, CH * k, M - CH * (k + 1))

            @pl.when(k < nc)
            def _():
                pltpu.make_async_remote_copy(
                    src_ref=sbuf_ref.at[pl.ds(o_send, CH), :],
                    dst_ref=rbuf_ref.at[pl.ds(o_send, CH), :],
                    send_sem=send_sems.at[k],
                    recv_sem=recv_sems.at[k],
                    device_id=(other, my_y),
                    device_id_type=pl.DeviceIdType.MESH,
                ).wait_send()

        for k in range(NCH):
            o_recv = jnp.where(my_x == 1, CH * k, M - CH * (k + 1))

            @pl.when(k < nc)
            def _():
                pltpu.make_async_remote_copy(
                    src_ref=sbuf_ref.at[pl.ds(o_recv, CH), :],
                    dst_ref=rbuf_ref.at[pl.ds(o_recv, CH), :],
                    send_sem=send_sems.at[k],
                    recv_sem=recv_sems.at[k],
                    device_id=(other, my_y),
                    device_id_type=pl.DeviceIdType.MESH,
                ).wait_recv()

        jo = lax.broadcasted_iota(jnp.int32, (M, 1), 0)
        in_keep = (jo >= off) & (jo < off + n_keep)
        out_ref[...] = jnp.where(in_keep, keep_rows, rbuf_ref[...])

    return pl.pallas_call(
        body,
        out_shape=jax.ShapeDtypeStruct((M, N), jnp.float32),
        in_specs=[
            pl.BlockSpec(memory_space=pltpu.VMEM),
            pl.BlockSpec(memory_space=pltpu.VMEM),
        ],
        out_specs=pl.BlockSpec(memory_space=pltpu.VMEM),
        scratch_shapes=[
            pltpu.VMEM((M, N), jnp.float32),
            pltpu.VMEM((M, N), jnp.float32),
            pltpu.SemaphoreType.DMA((NCH,)),
            pltpu.SemaphoreType.DMA((NCH,)),
        ],
        compiler_params=pltpu.CompilerParams(collective_id=0),
    )(x, dest2)
